# baseline (speedup 1.0000x reference)
import functools
import jax
import jax.numpy as jnp
from jax import lax
from jax.experimental import pallas as pl
from jax.experimental.pallas import tpu as pltpu
from jax.experimental.pallas import tpu_sc as plsc

N = 10000
E = 320000
D_IN = 128
D_H = 256
D_OUT = 128
G = 64

NC = 2
NS = 16
NW = NC * NS

E_PER_S = E // NS
CHUNK = 128
NBUF = 3
NCHUNK = NBUF * (-(-E_PER_S // (NBUF * CHUNK)))
E_PAD_S = NCHUNK * CHUNK

E_PER_W = E // NW

RCHUNK = 80
NRCHUNK = N // RCHUNK
RB_ITERS = -(-NRCHUNK // NS)

_mesh = plsc.VectorSubcoreMesh(core_axis_name="c", subcore_axis_name="s",
                               num_cores=NC, num_subcores=NS)
_sc_params = pltpu.CompilerParams(needs_layout_passes=False)
_HI = lax.Precision.HIGHEST


def _sc_deg_body(dst_hbm, degp_hbm, dstv, acc):
  c = lax.axis_index("c")
  s = lax.axis_index("s")
  wid = s * NC + c
  pltpu.sync_copy(dst_hbm.at[wid], dstv)

  zeros16 = jnp.zeros((16,), jnp.float32)
  ones16 = jnp.ones((16,), jnp.float32)

  def zero_body(i, carry):
    acc[pl.ds(i * 16, 16)] = zeros16
    return carry

  lax.fori_loop(0, N // 16, zero_body, 0, unroll=4)

  def add_body(i, carry):
    idx = dstv[pl.ds(i * 16, 16)]
    plsc.addupdate_scatter(acc, [idx], ones16)
    return carry

  lax.fori_loop(0, E_PER_W // 16, add_body, 0, unroll=4)
  pltpu.sync_copy(acc, degp_hbm.at[wid])


_sc_deg = pl.kernel(
    _sc_deg_body,
    out_type=jax.ShapeDtypeStruct((NW, N), jnp.float32),
    mesh=_mesh,
    compiler_params=_sc_params,
    scratch_types=[
        pltpu.VMEM((E_PER_W,), jnp.int32),
        pltpu.VMEM((N,), jnp.float32),
    ],
)


def _sc_agg_body(hlo, hhi, srcr, dstr, slo, shi, acc_sh, *bufs):
  c = lax.axis_index("c")
  s = lax.axis_index("s")
  srcis = bufs[0:NBUF]
  dstis = bufs[NBUF:2 * NBUF]
  rowss = bufs[2 * NBUF:3 * NBUF]
  gsems = bufs[3 * NBUF:4 * NBUF]
  isemss = bufs[4 * NBUF:5 * NBUF]
  isemds = bufs[5 * NBUF:6 * NBUF]

  def run(h_ref, out_ref):
    def init_body(t, carry):
      ch = s + t * NS

      @pl.when(ch < NRCHUNK)
      def _():
        r0 = pl.multiple_of(ch * RCHUNK, 8)
        pltpu.sync_copy(h_ref.at[pl.ds(r0, RCHUNK)], acc_sh.at[pl.ds(r0, RCHUNK)])

      return carry

    lax.fori_loop(0, RB_ITERS, init_body, 0)
    plsc.subcore_barrier()

    for k in range(NBUF):
      pltpu.sync_copy(srcr.at[s, k], srcis[k])
      pltpu.sync_copy(dstr.at[s, k], dstis[k])
      pltpu.async_copy(h_ref.at[srcis[k]], rowss[k], gsems[k])

    def grp_body(t, carry):
      def half(si, di, rbuf, gsem, isems, isemd, off):
        nxt = NBUF * t + NBUF + off
        pltpu.make_async_copy(h_ref.at[si], rbuf, gsem).wait()

        @pl.when(nxt < NCHUNK)
        def _():
          pltpu.async_copy(srcr.at[s, nxt], si, isems)

        @pl.when(t > 0)
        def _():
          pltpu.make_async_copy(dstr.at[s, 0], di, isemd).wait()

        pltpu.sync_copy(rbuf, acc_sh.at[di], add=True)

        @pl.when(nxt < NCHUNK)
        def _():
          pltpu.async_copy(dstr.at[s, nxt], di, isemd)
          pltpu.make_async_copy(srcr.at[s, 0], si, isems).wait()
          pltpu.async_copy(h_ref.at[si], rbuf, gsem)

      for k in range(NBUF):
        half(srcis[k], dstis[k], rowss[k], gsems[k], isemss[k], isemds[k], k)
      return carry

    lax.fori_loop(0, NCHUNK // NBUF, grp_body, 0)
    plsc.subcore_barrier()

    def out_body(t, carry):
      ch = s + t * NS

      @pl.when(ch < NRCHUNK)
      def _():
        r0 = pl.multiple_of(ch * RCHUNK, 8)
        pltpu.sync_copy(acc_sh.at[pl.ds(r0, RCHUNK)], out_ref.at[pl.ds(r0, RCHUNK)])

      return carry

    lax.fori_loop(0, RB_ITERS, out_body, 0)

  @pl.when(c == 0)
  def _():
    run(hlo, slo)

  @pl.when(c == 1)
  def _():
    run(hhi, shi)


_sc_agg = pl.kernel(
    _sc_agg_body,
    out_type=(
        jax.ShapeDtypeStruct((N, 128), jnp.float32),
        jax.ShapeDtypeStruct((N, 128), jnp.float32),
    ),
    mesh=_mesh,
    compiler_params=_sc_params,
    scratch_types=[
        pltpu.VMEM_SHARED((N + 8, 128), jnp.float32),
        *[pltpu.VMEM((CHUNK,), jnp.int32) for _ in range(2 * NBUF)],
        *[pltpu.VMEM((CHUNK, 128), jnp.float32) for _ in range(NBUF)],
        *[pltpu.SemaphoreType.DMA for _ in range(3 * NBUF)],
    ],
)


_BLK = 1000
_NBLK = N // _BLK


def _tc_layer1_body(degp_ref, x_ref, w1_ref, hlo_ref, hhi_ref, dinv_ref):
  deg = 1.0 + jnp.sum(degp_ref[...].reshape(NW, _BLK), axis=0)
  dinv = lax.rsqrt(deg)
  h = jnp.dot(x_ref[...], w1_ref[...], preferred_element_type=jnp.float32,
              precision=_HI)
  hs = h * dinv[:, None]
  hlo_ref[...] = hs[:, :128]
  hhi_ref[...] = hs[:, 128:]
  dinv_ref[...] = dinv[None, None, :]


def _tc_layer1(degp, x, w1):
  return pl.pallas_call(
      _tc_layer1_body,
      grid=(_NBLK,),
      in_specs=[
          pl.BlockSpec((NW, 1, 1, _BLK), lambda i: (0, i, 0, 0)),
          pl.BlockSpec((_BLK, D_IN), lambda i: (i, 0)),
          pl.BlockSpec((D_IN, D_H), lambda i: (0, 0)),
      ],
      out_specs=[
          pl.BlockSpec((_BLK, 128), lambda i: (i, 0)),
          pl.BlockSpec((_BLK, 128), lambda i: (i, 0)),
          pl.BlockSpec((1, 1, _BLK), lambda i: (i, 0, 0)),
      ],
      out_shape=[
          jax.ShapeDtypeStruct((N, 128), jnp.float32),
          jax.ShapeDtypeStruct((N, 128), jnp.float32),
          jax.ShapeDtypeStruct((_NBLK, 1, _BLK), jnp.float32),
      ],
  )(degp, x, w1)


def _tc_layer2_body(slo_ref, shi_ref, dinv_ref, b1_ref,
                    w2_ref, olo_ref, ohi_ref):
  st = jnp.concatenate([slo_ref[...], shi_ref[...]], axis=1)
  dinv = dinv_ref[0, 0, :]
  h1 = jnp.maximum(st * dinv[:, None] + b1_ref[0, :][None, :], 0.0)
  h2 = jnp.dot(h1, w2_ref[...], preferred_element_type=jnp.float32,
               precision=_HI) * dinv[:, None]
  olo_ref[...] = h2[:, :128]
  ohi_ref[...] = h2[:, 128:]


def _tc_layer2(slo, shi, dinv3, b1r, w2):
  blk = lambda i: (i, 0)
  return pl.pallas_call(
      _tc_layer2_body,
      grid=(_NBLK,),
      in_specs=[
          pl.BlockSpec((_BLK, 128), blk),
          pl.BlockSpec((_BLK, 128), blk),
          pl.BlockSpec((1, 1, _BLK), lambda i: (i, 0, 0)),
          pl.BlockSpec((1, D_H), lambda i: (0, 0)),
          pl.BlockSpec((D_H, D_H), lambda i: (0, 0)),
      ],
      out_specs=[
          pl.BlockSpec((_BLK, 128), blk),
          pl.BlockSpec((_BLK, 128), blk),
      ],
      out_shape=[
          jax.ShapeDtypeStruct((N, 128), jnp.float32),
          jax.ShapeDtypeStruct((N, 128), jnp.float32),
      ],
  )(slo, shi, dinv3, b1r, w2)


def _tc_pool_head_body(slo_ref, shi_ref, dinv_ref, b2_ref,
                       batch_ref, p1_ref, pb1_ref, p2_ref, pb2_ref, out_ref,
                       gsum, cnt):
  i = pl.program_id(0)

  @pl.when(i == 0)
  def _():
    gsum[...] = jnp.zeros((G, D_H), jnp.float32)
    cnt[...] = jnp.zeros((G, 8), jnp.float32)

  st = jnp.concatenate([slo_ref[...], shi_ref[...]], axis=1)
  dinv = dinv_ref[0, 0, :]
  h2 = jnp.maximum(st * dinv[:, None] + b2_ref[0, :][None, :], 0.0)
  b = batch_ref[0, 0, :]
  onehot = (lax.broadcasted_iota(jnp.int32, (G, _BLK), 0)
            == b[None, :]).astype(jnp.float32)
  gsum[...] += jnp.dot(onehot, h2, preferred_element_type=jnp.float32,
                       precision=_HI)
  cnt[...] += jnp.dot(onehot, jnp.ones((_BLK, 8), jnp.float32),
                      preferred_element_type=jnp.float32, precision=_HI)

  @pl.when(i == _NBLK - 1)
  def _():
    counts = cnt[:, 0:1]
    g = gsum[...] / jnp.maximum(counts, 1.0)
    r = jnp.maximum(
        jnp.dot(g, p1_ref[...], preferred_element_type=jnp.float32,
                precision=_HI) + pb1_ref[0, :][None, :], 0.0)
    out_ref[...] = jnp.dot(r, p2_ref[...], preferred_element_type=jnp.float32,
                           precision=_HI) + pb2_ref[0, :][None, :]


def _tc_pool_head(slo, shi, dinv3, b2r, batch3, p1, pb1r, p2, pb2r):
  blk = lambda i: (i, 0)
  return pl.pallas_call(
      _tc_pool_head_body,
      grid=(_NBLK,),
      in_specs=[
          pl.BlockSpec((_BLK, 128), blk),
          pl.BlockSpec((_BLK, 128), blk),
          pl.BlockSpec((1, 1, _BLK), lambda i: (i, 0, 0)),
          pl.BlockSpec((1, D_H), lambda i: (0, 0)),
          pl.BlockSpec((1, 1, _BLK), lambda i: (i, 0, 0)),
          pl.BlockSpec((D_H, D_H), lambda i: (0, 0)),
          pl.BlockSpec((1, D_H), lambda i: (0, 0)),
          pl.BlockSpec((D_H, D_OUT), lambda i: (0, 0)),
          pl.BlockSpec((1, D_OUT), lambda i: (0, 0)),
      ],
      out_specs=pl.BlockSpec((G, D_OUT), lambda i: (0, 0)),
      out_shape=jax.ShapeDtypeStruct((G, D_OUT), jnp.float32),
      scratch_shapes=[
          pltpu.VMEM((G, D_H), jnp.float32),
          pltpu.VMEM((G, 8), jnp.float32),
      ],
  )(slo, shi, dinv3, b2r, batch3, p1, pb1r, p2, pb2r)


@jax.jit
def kernel(x, edge_index, batch, W1, b1, W2, b2, P1, pb1, P2, pb2):
  src = edge_index[0].astype(jnp.int32)
  dst = edge_index[1].astype(jnp.int32)
  dstw = dst.reshape(NW, E_PER_W)
  pad = E_PAD_S - E_PER_S
  srcp = jnp.pad(src.reshape(NS, E_PER_S), ((0, 0), (0, pad)),
                 constant_values=0).reshape(NS, NCHUNK, CHUNK)
  dstp = jnp.pad(dst.reshape(NS, E_PER_S), ((0, 0), (0, pad)),
                 constant_values=N).reshape(NS, NCHUNK, CHUNK)
  batch3 = batch.astype(jnp.int32).reshape(_NBLK, 1, _BLK)
  b1r = b1.reshape(1, D_H)
  b2r = b2.reshape(1, D_H)
  pb1r = pb1.reshape(1, D_H)
  pb2r = pb2.reshape(1, D_OUT)

  degp = _sc_deg(dstw)
  degp4 = degp.reshape(NW, _NBLK, 1, _BLK)
  hlo, hhi, dinv3 = _tc_layer1(degp4, x, W1)
  slo, shi = _sc_agg(hlo, hhi, srcp, dstp)
  h2lo, h2hi = _tc_layer2(slo, shi, dinv3, b1r, W2)
  s2lo, s2hi = _sc_agg(h2lo, h2hi, srcp, dstp)
  return _tc_pool_head(s2lo, s2hi, dinv3, b2r, batch3,
                       P1, pb1r, P2, pb2r)


# speedup vs baseline: 1.1659x; 1.1659x over previous
import functools
import jax
import jax.numpy as jnp
from jax import lax
from jax.experimental import pallas as pl
from jax.experimental.pallas import tpu as pltpu
from jax.experimental.pallas import tpu_sc as plsc

N = 10000
E = 320000
D_IN = 128
D_H = 256
D_OUT = 128
G = 64

NC = 2
NS = 16
NW = NC * NS

E_PER_S = E // NS
CHUNK = 128
NBUF = 2
NCHUNK = NBUF * (-(-E_PER_S // (NBUF * CHUNK)))
E_PAD_S = NCHUNK * CHUNK

E_PER_W = E // NW

RCHUNK = 80
NRCHUNK = N // RCHUNK
RB_ITERS = -(-NRCHUNK // NS)

_mesh = plsc.VectorSubcoreMesh(core_axis_name="c", subcore_axis_name="s",
                               num_cores=NC, num_subcores=NS)
_sc_params = pltpu.CompilerParams(needs_layout_passes=False)
_HI = lax.Precision.HIGHEST


def _sc_deg_body(dst_hbm, degp_hbm, dstv, acc):
  c = lax.axis_index("c")
  s = lax.axis_index("s")
  wid = s * NC + c
  pltpu.sync_copy(dst_hbm.at[wid], dstv)

  zeros16 = jnp.zeros((16,), jnp.float32)
  ones16 = jnp.ones((16,), jnp.float32)

  def zero_body(i, carry):
    acc[pl.ds(i * 16, 16)] = zeros16
    return carry

  lax.fori_loop(0, N // 16, zero_body, 0, unroll=4)

  def add_body(i, carry):
    idx = dstv[pl.ds(i * 16, 16)]
    plsc.addupdate_scatter(acc, [idx], ones16)
    return carry

  lax.fori_loop(0, E_PER_W // 16, add_body, 0, unroll=4)
  pltpu.sync_copy(acc, degp_hbm.at[wid])


_sc_deg = pl.kernel(
    _sc_deg_body,
    out_type=jax.ShapeDtypeStruct((NW, N), jnp.float32),
    mesh=_mesh,
    compiler_params=_sc_params,
    scratch_types=[
        pltpu.VMEM((E_PER_W,), jnp.int32),
        pltpu.VMEM((N,), jnp.float32),
    ],
)


def _sc_agg_body(hlo, hhi, srcr, dstr, slo, shi, acc_sh, *bufs):
  c = lax.axis_index("c")
  s = lax.axis_index("s")
  srcis = bufs[0:NBUF]
  dstis = bufs[NBUF:2 * NBUF]
  rowss = bufs[2 * NBUF:3 * NBUF]
  gsems = bufs[3 * NBUF:4 * NBUF]
  isemss = bufs[4 * NBUF:5 * NBUF]
  isemds = bufs[5 * NBUF:6 * NBUF]

  def run(h_ref, out_ref):
    def init_body(t, carry):
      ch = s + t * NS

      @pl.when(ch < NRCHUNK)
      def _():
        r0 = pl.multiple_of(ch * RCHUNK, 8)
        pltpu.sync_copy(h_ref.at[pl.ds(r0, RCHUNK)], acc_sh.at[pl.ds(r0, RCHUNK)])

      return carry

    lax.fori_loop(0, RB_ITERS, init_body, 0)
    plsc.subcore_barrier()

    for k in range(NBUF):
      pltpu.sync_copy(srcr.at[s, k], srcis[k])
      pltpu.sync_copy(dstr.at[s, k], dstis[k])
      pltpu.async_copy(h_ref.at[srcis[k]], rowss[k], gsems[k])

    def grp_body(t, carry):
      def half(si, di, rbuf, gsem, isems, isemd, off):
        nxt = NBUF * t + NBUF + off
        pltpu.make_async_copy(h_ref.at[si], rbuf, gsem).wait()

        @pl.when(nxt < NCHUNK)
        def _():
          pltpu.async_copy(srcr.at[s, nxt], si, isems)

        @pl.when(t > 0)
        def _():
          pltpu.make_async_copy(dstr.at[s, 0], di, isemd).wait()

        pltpu.sync_copy(rbuf, acc_sh.at[di], add=True)

        @pl.when(nxt < NCHUNK)
        def _():
          pltpu.async_copy(dstr.at[s, nxt], di, isemd)
          pltpu.make_async_copy(srcr.at[s, 0], si, isems).wait()
          pltpu.async_copy(h_ref.at[si], rbuf, gsem)

      for k in range(NBUF):
        half(srcis[k], dstis[k], rowss[k], gsems[k], isemss[k], isemds[k], k)
      return carry

    lax.fori_loop(0, NCHUNK // NBUF, grp_body, 0)
    plsc.subcore_barrier()

    def out_body(t, carry):
      ch = s + t * NS

      @pl.when(ch < NRCHUNK)
      def _():
        r0 = pl.multiple_of(ch * RCHUNK, 8)
        pltpu.sync_copy(acc_sh.at[pl.ds(r0, RCHUNK)], out_ref.at[pl.ds(r0, RCHUNK)])

      return carry

    lax.fori_loop(0, RB_ITERS, out_body, 0)

  @pl.when(c == 0)
  def _():
    run(hlo, slo)

  @pl.when(c == 1)
  def _():
    run(hhi, shi)


_sc_agg = pl.kernel(
    _sc_agg_body,
    out_type=(
        jax.ShapeDtypeStruct((N, 128), jnp.float32),
        jax.ShapeDtypeStruct((N, 128), jnp.float32),
    ),
    mesh=_mesh,
    compiler_params=_sc_params,
    scratch_types=[
        pltpu.VMEM_SHARED((N + 8, 128), jnp.float32),
        *[pltpu.VMEM((CHUNK,), jnp.int32) for _ in range(2 * NBUF)],
        *[pltpu.VMEM((CHUNK, 128), jnp.float32) for _ in range(NBUF)],
        *[pltpu.SemaphoreType.DMA for _ in range(3 * NBUF)],
    ],
)


_BLK = 1000
_NBLK = N // _BLK


def _tc_layer1_body(degp_ref, x_ref, w1_ref, hlo_ref, hhi_ref, dinv_ref):
  deg = 1.0 + jnp.sum(degp_ref[...].reshape(NW, _BLK), axis=0)
  dinv = lax.rsqrt(deg)
  h = jnp.dot(x_ref[...], w1_ref[...], preferred_element_type=jnp.float32,
              precision=_HI)
  hs = h * dinv[:, None]
  hlo_ref[...] = hs[:, :128]
  hhi_ref[...] = hs[:, 128:]
  dinv_ref[...] = dinv[None, None, :]


def _tc_layer1(degp, x, w1):
  return pl.pallas_call(
      _tc_layer1_body,
      grid=(_NBLK,),
      in_specs=[
          pl.BlockSpec((NW, 1, 1, _BLK), lambda i: (0, i, 0, 0)),
          pl.BlockSpec((_BLK, D_IN), lambda i: (i, 0)),
          pl.BlockSpec((D_IN, D_H), lambda i: (0, 0)),
      ],
      out_specs=[
          pl.BlockSpec((_BLK, 128), lambda i: (i, 0)),
          pl.BlockSpec((_BLK, 128), lambda i: (i, 0)),
          pl.BlockSpec((1, 1, _BLK), lambda i: (i, 0, 0)),
      ],
      out_shape=[
          jax.ShapeDtypeStruct((N, 128), jnp.float32),
          jax.ShapeDtypeStruct((N, 128), jnp.float32),
          jax.ShapeDtypeStruct((_NBLK, 1, _BLK), jnp.float32),
      ],
  )(degp, x, w1)


def _tc_layer2_body(slo_ref, shi_ref, dinv_ref, b1_ref,
                    w2_ref, olo_ref, ohi_ref):
  st = jnp.concatenate([slo_ref[...], shi_ref[...]], axis=1)
  dinv = dinv_ref[0, 0, :]
  h1 = jnp.maximum(st * dinv[:, None] + b1_ref[0, :][None, :], 0.0)
  h2 = jnp.dot(h1, w2_ref[...], preferred_element_type=jnp.float32,
               precision=_HI) * dinv[:, None]
  olo_ref[...] = h2[:, :128]
  ohi_ref[...] = h2[:, 128:]


def _tc_layer2(slo, shi, dinv3, b1r, w2):
  blk = lambda i: (i, 0)
  return pl.pallas_call(
      _tc_layer2_body,
      grid=(_NBLK,),
      in_specs=[
          pl.BlockSpec((_BLK, 128), blk),
          pl.BlockSpec((_BLK, 128), blk),
          pl.BlockSpec((1, 1, _BLK), lambda i: (i, 0, 0)),
          pl.BlockSpec((1, D_H), lambda i: (0, 0)),
          pl.BlockSpec((D_H, D_H), lambda i: (0, 0)),
      ],
      out_specs=[
          pl.BlockSpec((_BLK, 128), blk),
          pl.BlockSpec((_BLK, 128), blk),
      ],
      out_shape=[
          jax.ShapeDtypeStruct((N, 128), jnp.float32),
          jax.ShapeDtypeStruct((N, 128), jnp.float32),
      ],
  )(slo, shi, dinv3, b1r, w2)


def _tc_pool_head_body(slo_ref, shi_ref, dinv_ref, b2_ref,
                       batch_ref, p1_ref, pb1_ref, p2_ref, pb2_ref, out_ref,
                       gsum, cnt):
  i = pl.program_id(0)

  @pl.when(i == 0)
  def _():
    gsum[...] = jnp.zeros((G, D_H), jnp.float32)
    cnt[...] = jnp.zeros((G, 8), jnp.float32)

  st = jnp.concatenate([slo_ref[...], shi_ref[...]], axis=1)
  dinv = dinv_ref[0, 0, :]
  h2 = jnp.maximum(st * dinv[:, None] + b2_ref[0, :][None, :], 0.0)
  b = batch_ref[0, 0, :]
  onehot = (lax.broadcasted_iota(jnp.int32, (G, _BLK), 0)
            == b[None, :]).astype(jnp.float32)
  gsum[...] += jnp.dot(onehot, h2, preferred_element_type=jnp.float32,
                       precision=_HI)
  cnt[...] += jnp.dot(onehot, jnp.ones((_BLK, 8), jnp.float32),
                      preferred_element_type=jnp.float32, precision=_HI)

  @pl.when(i == _NBLK - 1)
  def _():
    counts = cnt[:, 0:1]
    g = gsum[...] / jnp.maximum(counts, 1.0)
    r = jnp.maximum(
        jnp.dot(g, p1_ref[...], preferred_element_type=jnp.float32,
                precision=_HI) + pb1_ref[0, :][None, :], 0.0)
    out_ref[...] = jnp.dot(r, p2_ref[...], preferred_element_type=jnp.float32,
                           precision=_HI) + pb2_ref[0, :][None, :]


def _tc_pool_head(slo, shi, dinv3, b2r, batch3, p1, pb1r, p2, pb2r):
  blk = lambda i: (i, 0)
  return pl.pallas_call(
      _tc_pool_head_body,
      grid=(_NBLK,),
      in_specs=[
          pl.BlockSpec((_BLK, 128), blk),
          pl.BlockSpec((_BLK, 128), blk),
          pl.BlockSpec((1, 1, _BLK), lambda i: (i, 0, 0)),
          pl.BlockSpec((1, D_H), lambda i: (0, 0)),
          pl.BlockSpec((1, 1, _BLK), lambda i: (i, 0, 0)),
          pl.BlockSpec((D_H, D_H), lambda i: (0, 0)),
          pl.BlockSpec((1, D_H), lambda i: (0, 0)),
          pl.BlockSpec((D_H, D_OUT), lambda i: (0, 0)),
          pl.BlockSpec((1, D_OUT), lambda i: (0, 0)),
      ],
      out_specs=pl.BlockSpec((G, D_OUT), lambda i: (0, 0)),
      out_shape=jax.ShapeDtypeStruct((G, D_OUT), jnp.float32),
      scratch_shapes=[
          pltpu.VMEM((G, D_H), jnp.float32),
          pltpu.VMEM((G, 8), jnp.float32),
      ],
  )(slo, shi, dinv3, b2r, batch3, p1, pb1r, p2, pb2r)


@jax.jit
def kernel(x, edge_index, batch, W1, b1, W2, b2, P1, pb1, P2, pb2):
  src = edge_index[0].astype(jnp.int32)
  dst = edge_index[1].astype(jnp.int32)
  dstw = dst.reshape(NW, E_PER_W)
  pad = E_PAD_S - E_PER_S
  srcp = jnp.pad(src.reshape(NS, E_PER_S), ((0, 0), (0, pad)),
                 constant_values=0).reshape(NS, NCHUNK, CHUNK)
  dstp = jnp.pad(dst.reshape(NS, E_PER_S), ((0, 0), (0, pad)),
                 constant_values=N).reshape(NS, NCHUNK, CHUNK)
  batch3 = batch.astype(jnp.int32).reshape(_NBLK, 1, _BLK)
  b1r = b1.reshape(1, D_H)
  b2r = b2.reshape(1, D_H)
  pb1r = pb1.reshape(1, D_H)
  pb2r = pb2.reshape(1, D_OUT)

  degp = _sc_deg(dstw)
  degp4 = degp.reshape(NW, _NBLK, 1, _BLK)
  hlo, hhi, dinv3 = _tc_layer1(degp4, x, W1)
  slo, shi = _sc_agg(hlo, hhi, srcp, dstp)
  h2lo, h2hi = _tc_layer2(slo, shi, dinv3, b1r, W2)
  s2lo, s2hi = _sc_agg(h2lo, h2hi, srcp, dstp)
  return _tc_pool_head(s2lo, s2hi, dinv3, b2r, batch3,
                       P1, pb1r, P2, pb2r)


# speedup vs baseline: 1.1663x; 1.0003x over previous
import functools
import jax
import jax.numpy as jnp
from jax import lax
from jax.experimental import pallas as pl
from jax.experimental.pallas import tpu as pltpu
from jax.experimental.pallas import tpu_sc as plsc

N = 10000
E = 320000
D_IN = 128
D_H = 256
D_OUT = 128
G = 64

NC = 2
NS = 16
NW = NC * NS

E_PER_S = E // NS
CHUNK = 128
NBUF = 2
NCHUNK = NBUF * (-(-E_PER_S // (NBUF * CHUNK)))
E_PAD_S = NCHUNK * CHUNK

E_PER_W = E // NW

RCHUNK = 80
NRCHUNK = N // RCHUNK
RB_ITERS = -(-NRCHUNK // NS)

_mesh = plsc.VectorSubcoreMesh(core_axis_name="c", subcore_axis_name="s",
                               num_cores=NC, num_subcores=NS)
_sc_params = pltpu.CompilerParams(needs_layout_passes=False)
_HI = lax.Precision.HIGHEST


def _sc_deg_body(dst_hbm, degp_hbm, dstv, acc):
  c = lax.axis_index("c")
  s = lax.axis_index("s")
  wid = s * NC + c
  pltpu.sync_copy(dst_hbm.at[wid], dstv)

  zeros16 = jnp.zeros((16,), jnp.float32)
  ones16 = jnp.ones((16,), jnp.float32)

  def zero_body(i, carry):
    acc[pl.ds(i * 16, 16)] = zeros16
    return carry

  lax.fori_loop(0, N // 16, zero_body, 0, unroll=4)

  def add_body(i, carry):
    idx = dstv[pl.ds(i * 16, 16)]
    plsc.addupdate_scatter(acc, [idx], ones16)
    return carry

  lax.fori_loop(0, E_PER_W // 16, add_body, 0, unroll=4)
  pltpu.sync_copy(acc, degp_hbm.at[wid])


_sc_deg = pl.kernel(
    _sc_deg_body,
    out_type=jax.ShapeDtypeStruct((NW, N), jnp.float32),
    mesh=_mesh,
    compiler_params=_sc_params,
    scratch_types=[
        pltpu.VMEM((E_PER_W,), jnp.int32),
        pltpu.VMEM((N,), jnp.float32),
    ],
)


def _sc_agg_body(hlo, hhi, srcr, dstr, slo, shi, acc_sh, *bufs):
  c = lax.axis_index("c")
  s = lax.axis_index("s")
  srcis = bufs[0:NBUF]
  dstis = bufs[NBUF:2 * NBUF]
  rowss = bufs[2 * NBUF:3 * NBUF]
  gsems = bufs[3 * NBUF:4 * NBUF]
  isemss = bufs[4 * NBUF:5 * NBUF]
  isemds = bufs[5 * NBUF:6 * NBUF]

  def run(h_ref, out_ref):
    def init_body(t, carry):
      ch = s + t * NS

      @pl.when(ch < NRCHUNK)
      def _():
        r0 = pl.multiple_of(ch * RCHUNK, 8)
        pltpu.sync_copy(h_ref.at[pl.ds(r0, RCHUNK)], acc_sh.at[pl.ds(r0, RCHUNK)])

      return carry

    lax.fori_loop(0, RB_ITERS, init_body, 0)
    plsc.subcore_barrier()

    HC = CHUNK // 2

    def start_gather(si, rbuf, gsem):
      pltpu.async_copy(h_ref.at[si.at[pl.ds(0, HC)]], rbuf.at[pl.ds(0, HC)],
                       gsem)
      pltpu.async_copy(h_ref.at[si.at[pl.ds(HC, HC)]], rbuf.at[pl.ds(HC, HC)],
                       gsem)

    for k in range(NBUF):
      pltpu.sync_copy(srcr.at[s, k], srcis[k])
      pltpu.sync_copy(dstr.at[s, k], dstis[k])
      start_gather(srcis[k], rowss[k], gsems[k])

    def grp_body(t, carry):
      def half(si, di, rbuf, gsem, isems, isemd, off):
        nxt = NBUF * t + NBUF + off
        pltpu.make_async_copy(h_ref.at[si], rbuf, gsem).wait()

        @pl.when(nxt < NCHUNK)
        def _():
          pltpu.async_copy(srcr.at[s, nxt], si, isems)

        @pl.when(t > 0)
        def _():
          pltpu.make_async_copy(dstr.at[s, 0], di, isemd).wait()

        pltpu.sync_copy(rbuf, acc_sh.at[di], add=True)

        @pl.when(nxt < NCHUNK)
        def _():
          pltpu.async_copy(dstr.at[s, nxt], di, isemd)
          pltpu.make_async_copy(srcr.at[s, 0], si, isems).wait()
          start_gather(si, rbuf, gsem)

      for k in range(NBUF):
        half(srcis[k], dstis[k], rowss[k], gsems[k], isemss[k], isemds[k], k)
      return carry

    lax.fori_loop(0, NCHUNK // NBUF, grp_body, 0)
    plsc.subcore_barrier()

    def out_body(t, carry):
      ch = s + t * NS

      @pl.when(ch < NRCHUNK)
      def _():
        r0 = pl.multiple_of(ch * RCHUNK, 8)
        pltpu.sync_copy(acc_sh.at[pl.ds(r0, RCHUNK)], out_ref.at[pl.ds(r0, RCHUNK)])

      return carry

    lax.fori_loop(0, RB_ITERS, out_body, 0)

  @pl.when(c == 0)
  def _():
    run(hlo, slo)

  @pl.when(c == 1)
  def _():
    run(hhi, shi)


_sc_agg = pl.kernel(
    _sc_agg_body,
    out_type=(
        jax.ShapeDtypeStruct((N, 128), jnp.float32),
        jax.ShapeDtypeStruct((N, 128), jnp.float32),
    ),
    mesh=_mesh,
    compiler_params=_sc_params,
    scratch_types=[
        pltpu.VMEM_SHARED((N + 8, 128), jnp.float32),
        *[pltpu.VMEM((CHUNK,), jnp.int32) for _ in range(2 * NBUF)],
        *[pltpu.VMEM((CHUNK, 128), jnp.float32) for _ in range(NBUF)],
        *[pltpu.SemaphoreType.DMA for _ in range(3 * NBUF)],
    ],
)


_BLK = 1000
_NBLK = N // _BLK


def _tc_layer1_body(degp_ref, x_ref, w1_ref, hlo_ref, hhi_ref, dinv_ref):
  deg = 1.0 + jnp.sum(degp_ref[...].reshape(NW, _BLK), axis=0)
  dinv = lax.rsqrt(deg)
  h = jnp.dot(x_ref[...], w1_ref[...], preferred_element_type=jnp.float32,
              precision=_HI)
  hs = h * dinv[:, None]
  hlo_ref[...] = hs[:, :128]
  hhi_ref[...] = hs[:, 128:]
  dinv_ref[...] = dinv[None, None, :]


def _tc_layer1(degp, x, w1):
  return pl.pallas_call(
      _tc_layer1_body,
      grid=(_NBLK,),
      in_specs=[
          pl.BlockSpec((NW, 1, 1, _BLK), lambda i: (0, i, 0, 0)),
          pl.BlockSpec((_BLK, D_IN), lambda i: (i, 0)),
          pl.BlockSpec((D_IN, D_H), lambda i: (0, 0)),
      ],
      out_specs=[
          pl.BlockSpec((_BLK, 128), lambda i: (i, 0)),
          pl.BlockSpec((_BLK, 128), lambda i: (i, 0)),
          pl.BlockSpec((1, 1, _BLK), lambda i: (i, 0, 0)),
      ],
      out_shape=[
          jax.ShapeDtypeStruct((N, 128), jnp.float32),
          jax.ShapeDtypeStruct((N, 128), jnp.float32),
          jax.ShapeDtypeStruct((_NBLK, 1, _BLK), jnp.float32),
      ],
  )(degp, x, w1)


def _tc_layer2_body(slo_ref, shi_ref, dinv_ref, b1_ref,
                    w2_ref, olo_ref, ohi_ref):
  st = jnp.concatenate([slo_ref[...], shi_ref[...]], axis=1)
  dinv = dinv_ref[0, 0, :]
  h1 = jnp.maximum(st * dinv[:, None] + b1_ref[0, :][None, :], 0.0)
  h2 = jnp.dot(h1, w2_ref[...], preferred_element_type=jnp.float32,
               precision=_HI) * dinv[:, None]
  olo_ref[...] = h2[:, :128]
  ohi_ref[...] = h2[:, 128:]


def _tc_layer2(slo, shi, dinv3, b1r, w2):
  blk = lambda i: (i, 0)
  return pl.pallas_call(
      _tc_layer2_body,
      grid=(_NBLK,),
      in_specs=[
          pl.BlockSpec((_BLK, 128), blk),
          pl.BlockSpec((_BLK, 128), blk),
          pl.BlockSpec((1, 1, _BLK), lambda i: (i, 0, 0)),
          pl.BlockSpec((1, D_H), lambda i: (0, 0)),
          pl.BlockSpec((D_H, D_H), lambda i: (0, 0)),
      ],
      out_specs=[
          pl.BlockSpec((_BLK, 128), blk),
          pl.BlockSpec((_BLK, 128), blk),
      ],
      out_shape=[
          jax.ShapeDtypeStruct((N, 128), jnp.float32),
          jax.ShapeDtypeStruct((N, 128), jnp.float32),
      ],
  )(slo, shi, dinv3, b1r, w2)


def _tc_pool_head_body(slo_ref, shi_ref, dinv_ref, b2_ref,
                       batch_ref, p1_ref, pb1_ref, p2_ref, pb2_ref, out_ref,
                       gsum, cnt):
  i = pl.program_id(0)

  @pl.when(i == 0)
  def _():
    gsum[...] = jnp.zeros((G, D_H), jnp.float32)
    cnt[...] = jnp.zeros((G, 8), jnp.float32)

  st = jnp.concatenate([slo_ref[...], shi_ref[...]], axis=1)
  dinv = dinv_ref[0, 0, :]
  h2 = jnp.maximum(st * dinv[:, None] + b2_ref[0, :][None, :], 0.0)
  b = batch_ref[0, 0, :]
  onehot = (lax.broadcasted_iota(jnp.int32, (G, _BLK), 0)
            == b[None, :]).astype(jnp.float32)
  gsum[...] += jnp.dot(onehot, h2, preferred_element_type=jnp.float32,
                       precision=_HI)
  cnt[...] += jnp.dot(onehot, jnp.ones((_BLK, 8), jnp.float32),
                      preferred_element_type=jnp.float32, precision=_HI)

  @pl.when(i == _NBLK - 1)
  def _():
    counts = cnt[:, 0:1]
    g = gsum[...] / jnp.maximum(counts, 1.0)
    r = jnp.maximum(
        jnp.dot(g, p1_ref[...], preferred_element_type=jnp.float32,
                precision=_HI) + pb1_ref[0, :][None, :], 0.0)
    out_ref[...] = jnp.dot(r, p2_ref[...], preferred_element_type=jnp.float32,
                           precision=_HI) + pb2_ref[0, :][None, :]


def _tc_pool_head(slo, shi, dinv3, b2r, batch3, p1, pb1r, p2, pb2r):
  blk = lambda i: (i, 0)
  return pl.pallas_call(
      _tc_pool_head_body,
      grid=(_NBLK,),
      in_specs=[
          pl.BlockSpec((_BLK, 128), blk),
          pl.BlockSpec((_BLK, 128), blk),
          pl.BlockSpec((1, 1, _BLK), lambda i: (i, 0, 0)),
          pl.BlockSpec((1, D_H), lambda i: (0, 0)),
          pl.BlockSpec((1, 1, _BLK), lambda i: (i, 0, 0)),
          pl.BlockSpec((D_H, D_H), lambda i: (0, 0)),
          pl.BlockSpec((1, D_H), lambda i: (0, 0)),
          pl.BlockSpec((D_H, D_OUT), lambda i: (0, 0)),
          pl.BlockSpec((1, D_OUT), lambda i: (0, 0)),
      ],
      out_specs=pl.BlockSpec((G, D_OUT), lambda i: (0, 0)),
      out_shape=jax.ShapeDtypeStruct((G, D_OUT), jnp.float32),
      scratch_shapes=[
          pltpu.VMEM((G, D_H), jnp.float32),
          pltpu.VMEM((G, 8), jnp.float32),
      ],
  )(slo, shi, dinv3, b2r, batch3, p1, pb1r, p2, pb2r)


@jax.jit
def kernel(x, edge_index, batch, W1, b1, W2, b2, P1, pb1, P2, pb2):
  src = edge_index[0].astype(jnp.int32)
  dst = edge_index[1].astype(jnp.int32)
  dstw = dst.reshape(NW, E_PER_W)
  pad = E_PAD_S - E_PER_S
  srcp = jnp.pad(src.reshape(NS, E_PER_S), ((0, 0), (0, pad)),
                 constant_values=0).reshape(NS, NCHUNK, CHUNK)
  dstp = jnp.pad(dst.reshape(NS, E_PER_S), ((0, 0), (0, pad)),
                 constant_values=N).reshape(NS, NCHUNK, CHUNK)
  batch3 = batch.astype(jnp.int32).reshape(_NBLK, 1, _BLK)
  b1r = b1.reshape(1, D_H)
  b2r = b2.reshape(1, D_H)
  pb1r = pb1.reshape(1, D_H)
  pb2r = pb2.reshape(1, D_OUT)

  degp = _sc_deg(dstw)
  degp4 = degp.reshape(NW, _NBLK, 1, _BLK)
  hlo, hhi, dinv3 = _tc_layer1(degp4, x, W1)
  slo, shi = _sc_agg(hlo, hhi, srcp, dstp)
  h2lo, h2hi = _tc_layer2(slo, shi, dinv3, b1r, W2)
  s2lo, s2hi = _sc_agg(h2lo, h2hi, srcp, dstp)
  return _tc_pool_head(s2lo, s2hi, dinv3, b2r, batch3,
                       P1, pb1r, P2, pb2r)


# speedup vs baseline: 1.1825x; 1.0139x over previous
import functools
import jax
import jax.numpy as jnp
from jax import lax
from jax.experimental import pallas as pl
from jax.experimental.pallas import tpu as pltpu
from jax.experimental.pallas import tpu_sc as plsc

N = 10000
E = 320000
D_IN = 128
D_H = 256
D_OUT = 128
G = 64

NC = 2
NS = 16
NW = NC * NS

E_PER_S = E // NS
CHUNK = 128
NBUF = 2
NCHUNK = NBUF * (-(-E_PER_S // (NBUF * CHUNK)))
E_PAD_S = NCHUNK * CHUNK

E_PER_W = E // NW

RCHUNK = 80
NRCHUNK = N // RCHUNK
RB_ITERS = -(-NRCHUNK // NS)

_mesh = plsc.VectorSubcoreMesh(core_axis_name="c", subcore_axis_name="s",
                               num_cores=NC, num_subcores=NS)
_sc_params = pltpu.CompilerParams(needs_layout_passes=False)
_HI = lax.Precision.DEFAULT


def _sc_deg_body(dst_hbm, degp_hbm, dstv, acc):
  c = lax.axis_index("c")
  s = lax.axis_index("s")
  wid = s * NC + c
  pltpu.sync_copy(dst_hbm.at[wid], dstv)

  zeros16 = jnp.zeros((16,), jnp.float32)
  ones16 = jnp.ones((16,), jnp.float32)

  def zero_body(i, carry):
    acc[pl.ds(i * 16, 16)] = zeros16
    return carry

  lax.fori_loop(0, N // 16, zero_body, 0, unroll=4)

  def add_body(i, carry):
    idx = dstv[pl.ds(i * 16, 16)]
    plsc.addupdate_scatter(acc, [idx], ones16)
    return carry

  lax.fori_loop(0, E_PER_W // 16, add_body, 0, unroll=4)
  pltpu.sync_copy(acc, degp_hbm.at[wid])


_sc_deg = pl.kernel(
    _sc_deg_body,
    out_type=jax.ShapeDtypeStruct((NW, N), jnp.float32),
    mesh=_mesh,
    compiler_params=_sc_params,
    scratch_types=[
        pltpu.VMEM((E_PER_W,), jnp.int32),
        pltpu.VMEM((N,), jnp.float32),
    ],
)


def _sc_agg_body(hlo, hhi, srcr, dstr, slo, shi, acc_sh, *bufs):
  c = lax.axis_index("c")
  s = lax.axis_index("s")
  srcis = bufs[0:NBUF]
  dstis = bufs[NBUF:2 * NBUF]
  rowss = bufs[2 * NBUF:3 * NBUF]
  gsems = bufs[3 * NBUF:4 * NBUF]
  isemss = bufs[4 * NBUF:5 * NBUF]
  isemds = bufs[5 * NBUF:6 * NBUF]

  def run(h_ref, out_ref):
    def init_body(t, carry):
      ch = s + t * NS

      @pl.when(ch < NRCHUNK)
      def _():
        r0 = pl.multiple_of(ch * RCHUNK, 8)
        pltpu.sync_copy(h_ref.at[pl.ds(r0, RCHUNK)], acc_sh.at[pl.ds(r0, RCHUNK)])

      return carry

    lax.fori_loop(0, RB_ITERS, init_body, 0)
    plsc.subcore_barrier()

    HC = CHUNK // 2

    def start_gather(si, rbuf, gsem):
      pltpu.async_copy(h_ref.at[si.at[pl.ds(0, HC)]], rbuf.at[pl.ds(0, HC)],
                       gsem)
      pltpu.async_copy(h_ref.at[si.at[pl.ds(HC, HC)]], rbuf.at[pl.ds(HC, HC)],
                       gsem)

    for k in range(NBUF):
      pltpu.sync_copy(srcr.at[s, k], srcis[k])
      pltpu.sync_copy(dstr.at[s, k], dstis[k])
      start_gather(srcis[k], rowss[k], gsems[k])

    def grp_body(t, carry):
      def half(si, di, rbuf, gsem, isems, isemd, off):
        nxt = NBUF * t + NBUF + off
        pltpu.make_async_copy(h_ref.at[si], rbuf, gsem).wait()

        @pl.when(nxt < NCHUNK)
        def _():
          pltpu.async_copy(srcr.at[s, nxt], si, isems)

        @pl.when(t > 0)
        def _():
          pltpu.make_async_copy(dstr.at[s, 0], di, isemd).wait()

        pltpu.sync_copy(rbuf, acc_sh.at[di], add=True)

        @pl.when(nxt < NCHUNK)
        def _():
          pltpu.async_copy(dstr.at[s, nxt], di, isemd)
          pltpu.make_async_copy(srcr.at[s, 0], si, isems).wait()
          start_gather(si, rbuf, gsem)

      for k in range(NBUF):
        half(srcis[k], dstis[k], rowss[k], gsems[k], isemss[k], isemds[k], k)
      return carry

    lax.fori_loop(0, NCHUNK // NBUF, grp_body, 0)
    plsc.subcore_barrier()

    def out_body(t, carry):
      ch = s + t * NS

      @pl.when(ch < NRCHUNK)
      def _():
        r0 = pl.multiple_of(ch * RCHUNK, 8)
        pltpu.sync_copy(acc_sh.at[pl.ds(r0, RCHUNK)], out_ref.at[pl.ds(r0, RCHUNK)])

      return carry

    lax.fori_loop(0, RB_ITERS, out_body, 0)

  @pl.when(c == 0)
  def _():
    run(hlo, slo)

  @pl.when(c == 1)
  def _():
    run(hhi, shi)


_sc_agg = pl.kernel(
    _sc_agg_body,
    out_type=(
        jax.ShapeDtypeStruct((N, 128), jnp.float32),
        jax.ShapeDtypeStruct((N, 128), jnp.float32),
    ),
    mesh=_mesh,
    compiler_params=_sc_params,
    scratch_types=[
        pltpu.VMEM_SHARED((N + 8, 128), jnp.float32),
        *[pltpu.VMEM((CHUNK,), jnp.int32) for _ in range(2 * NBUF)],
        *[pltpu.VMEM((CHUNK, 128), jnp.float32) for _ in range(NBUF)],
        *[pltpu.SemaphoreType.DMA for _ in range(3 * NBUF)],
    ],
)


_BLK = 1000
_NBLK = N // _BLK


def _tc_layer1_body(degp_ref, x_ref, w1_ref, hlo_ref, hhi_ref, dinv_ref):
  deg = 1.0 + jnp.sum(degp_ref[...].reshape(NW, _BLK), axis=0)
  dinv = lax.rsqrt(deg)
  h = jnp.dot(x_ref[...], w1_ref[...], preferred_element_type=jnp.float32,
              precision=_HI)
  hs = h * dinv[:, None]
  hlo_ref[...] = hs[:, :128]
  hhi_ref[...] = hs[:, 128:]
  dinv_ref[...] = dinv[None, None, :]


def _tc_layer1(degp, x, w1):
  return pl.pallas_call(
      _tc_layer1_body,
      grid=(_NBLK,),
      in_specs=[
          pl.BlockSpec((NW, 1, 1, _BLK), lambda i: (0, i, 0, 0)),
          pl.BlockSpec((_BLK, D_IN), lambda i: (i, 0)),
          pl.BlockSpec((D_IN, D_H), lambda i: (0, 0)),
      ],
      out_specs=[
          pl.BlockSpec((_BLK, 128), lambda i: (i, 0)),
          pl.BlockSpec((_BLK, 128), lambda i: (i, 0)),
          pl.BlockSpec((1, 1, _BLK), lambda i: (i, 0, 0)),
      ],
      out_shape=[
          jax.ShapeDtypeStruct((N, 128), jnp.float32),
          jax.ShapeDtypeStruct((N, 128), jnp.float32),
          jax.ShapeDtypeStruct((_NBLK, 1, _BLK), jnp.float32),
      ],
  )(degp, x, w1)


def _tc_layer2_body(slo_ref, shi_ref, dinv_ref, b1_ref,
                    w2_ref, olo_ref, ohi_ref):
  st = jnp.concatenate([slo_ref[...], shi_ref[...]], axis=1)
  dinv = dinv_ref[0, 0, :]
  h1 = jnp.maximum(st * dinv[:, None] + b1_ref[0, :][None, :], 0.0)
  h2 = jnp.dot(h1, w2_ref[...], preferred_element_type=jnp.float32,
               precision=_HI) * dinv[:, None]
  olo_ref[...] = h2[:, :128]
  ohi_ref[...] = h2[:, 128:]


def _tc_layer2(slo, shi, dinv3, b1r, w2):
  blk = lambda i: (i, 0)
  return pl.pallas_call(
      _tc_layer2_body,
      grid=(_NBLK,),
      in_specs=[
          pl.BlockSpec((_BLK, 128), blk),
          pl.BlockSpec((_BLK, 128), blk),
          pl.BlockSpec((1, 1, _BLK), lambda i: (i, 0, 0)),
          pl.BlockSpec((1, D_H), lambda i: (0, 0)),
          pl.BlockSpec((D_H, D_H), lambda i: (0, 0)),
      ],
      out_specs=[
          pl.BlockSpec((_BLK, 128), blk),
          pl.BlockSpec((_BLK, 128), blk),
      ],
      out_shape=[
          jax.ShapeDtypeStruct((N, 128), jnp.float32),
          jax.ShapeDtypeStruct((N, 128), jnp.float32),
      ],
  )(slo, shi, dinv3, b1r, w2)


def _tc_pool_head_body(slo_ref, shi_ref, dinv_ref, b2_ref,
                       batch_ref, p1_ref, pb1_ref, p2_ref, pb2_ref, out_ref,
                       gsum, cnt):
  i = pl.program_id(0)

  @pl.when(i == 0)
  def _():
    gsum[...] = jnp.zeros((G, D_H), jnp.float32)
    cnt[...] = jnp.zeros((G, 8), jnp.float32)

  st = jnp.concatenate([slo_ref[...], shi_ref[...]], axis=1)
  dinv = dinv_ref[0, 0, :]
  h2 = jnp.maximum(st * dinv[:, None] + b2_ref[0, :][None, :], 0.0)
  b = batch_ref[0, 0, :]
  onehot = (lax.broadcasted_iota(jnp.int32, (G, _BLK), 0)
            == b[None, :]).astype(jnp.float32)
  gsum[...] += jnp.dot(onehot, h2, preferred_element_type=jnp.float32,
                       precision=_HI)
  cnt[...] += jnp.dot(onehot, jnp.ones((_BLK, 8), jnp.float32),
                      preferred_element_type=jnp.float32, precision=_HI)

  @pl.when(i == _NBLK - 1)
  def _():
    counts = cnt[:, 0:1]
    g = gsum[...] / jnp.maximum(counts, 1.0)
    r = jnp.maximum(
        jnp.dot(g, p1_ref[...], preferred_element_type=jnp.float32,
                precision=_HI) + pb1_ref[0, :][None, :], 0.0)
    out_ref[...] = jnp.dot(r, p2_ref[...], preferred_element_type=jnp.float32,
                           precision=_HI) + pb2_ref[0, :][None, :]


def _tc_pool_head(slo, shi, dinv3, b2r, batch3, p1, pb1r, p2, pb2r):
  blk = lambda i: (i, 0)
  return pl.pallas_call(
      _tc_pool_head_body,
      grid=(_NBLK,),
      in_specs=[
          pl.BlockSpec((_BLK, 128), blk),
          pl.BlockSpec((_BLK, 128), blk),
          pl.BlockSpec((1, 1, _BLK), lambda i: (i, 0, 0)),
          pl.BlockSpec((1, D_H), lambda i: (0, 0)),
          pl.BlockSpec((1, 1, _BLK), lambda i: (i, 0, 0)),
          pl.BlockSpec((D_H, D_H), lambda i: (0, 0)),
          pl.BlockSpec((1, D_H), lambda i: (0, 0)),
          pl.BlockSpec((D_H, D_OUT), lambda i: (0, 0)),
          pl.BlockSpec((1, D_OUT), lambda i: (0, 0)),
      ],
      out_specs=pl.BlockSpec((G, D_OUT), lambda i: (0, 0)),
      out_shape=jax.ShapeDtypeStruct((G, D_OUT), jnp.float32),
      scratch_shapes=[
          pltpu.VMEM((G, D_H), jnp.float32),
          pltpu.VMEM((G, 8), jnp.float32),
      ],
  )(slo, shi, dinv3, b2r, batch3, p1, pb1r, p2, pb2r)


@jax.jit
def kernel(x, edge_index, batch, W1, b1, W2, b2, P1, pb1, P2, pb2):
  src = edge_index[0].astype(jnp.int32)
  dst = edge_index[1].astype(jnp.int32)
  dstw = dst.reshape(NW, E_PER_W)
  pad = E_PAD_S - E_PER_S
  srcp = jnp.pad(src.reshape(NS, E_PER_S), ((0, 0), (0, pad)),
                 constant_values=0).reshape(NS, NCHUNK, CHUNK)
  dstp = jnp.pad(dst.reshape(NS, E_PER_S), ((0, 0), (0, pad)),
                 constant_values=N).reshape(NS, NCHUNK, CHUNK)
  batch3 = batch.astype(jnp.int32).reshape(_NBLK, 1, _BLK)
  b1r = b1.reshape(1, D_H)
  b2r = b2.reshape(1, D_H)
  pb1r = pb1.reshape(1, D_H)
  pb2r = pb2.reshape(1, D_OUT)

  degp = _sc_deg(dstw)
  degp4 = degp.reshape(NW, _NBLK, 1, _BLK)
  hlo, hhi, dinv3 = _tc_layer1(degp4, x, W1)
  slo, shi = _sc_agg(hlo, hhi, srcp, dstp)
  h2lo, h2hi = _tc_layer2(slo, shi, dinv3, b1r, W2)
  s2lo, s2hi = _sc_agg(h2lo, h2hi, srcp, dstp)
  return _tc_pool_head(s2lo, s2hi, dinv3, b2r, batch3,
                       P1, pb1r, P2, pb2r)
